# baseline (speedup 1.0000x reference)
import functools

import jax
import jax.numpy as jnp
from jax import lax
from jax.experimental import pallas as pl
from jax.experimental.pallas import tpu as pltpu
from jax.experimental.pallas import tpu_sc as plsc

RADIUS = 0.1
NSAMPLE = 32
DENSITY_K = 8
EPS = 1e-08
BN_EPS = 1e-05

N = 2048
C = 128
RB = 256
CH = 64
HIGHEST = jax.lax.Precision.HIGHEST


def _tmin0(x):
    s = x.shape[0]
    while s > 8:
        h = s // 2
        x = jnp.minimum(x[:h], x[h:s])
        s = h
    return jnp.min(x, axis=0, keepdims=True)


def _tsum0(x):
    s = x.shape[0]
    while s > 8:
        h = s // 2
        x = x[:h] + x[h:s]
        s = h
    return jnp.sum(x, axis=0, keepdims=True)


def _table_kernel(feats_ref, coords_ref, wf_ref, wrp_ref, scale_ref, t_ref):
    f = feats_ref[0]
    cf = coords_ref[0]
    wf = wf_ref[...]
    wrp = wrp_ref[...]
    scale = scale_ref[...]
    g = jax.lax.dot_general(f, wf, (((0,), (1,)), ((), ())),
                            preferred_element_type=jnp.float32)
    cfw = jax.lax.dot_general(cf * 10.0, wrp, (((1,), (0,)), ((), ())),
                              preferred_element_type=jnp.float32)
    t_ref[0] = (g + cfw) * scale


def _sel_kernel(ctb_ref, cfull_ref, idx_ref, w_ref, d2_ref, gct_ref,
                idxf_ref):
    bf = pl.program_id(0).astype(jnp.float32)
    ctb = ctb_ref[0]
    cf = cfull_ref[0]

    nbt = (ctb[0:1, :] * ctb[0:1, :] + ctb[1:2, :] * ctb[1:2, :]
           + ctb[2:3, :] * ctb[2:3, :])
    cfx = cf[:, 0:1]
    cfy = cf[:, 1:2]
    cfz = cf[:, 2:3]
    nft = cfx * cfx + cfy * cfy + cfz * cfz
    crosst = jax.lax.dot_general(cf, ctb, (((1,), (0,)), ((), ())),
                                 preferred_element_type=jnp.float32)
    d2_ref[...] = jnp.clip((nbt + nft) - 2.0 * crosst, 1e-12, None)

    iota0 = jax.lax.broadcasted_iota(jnp.int32, (N, RB), 0).astype(jnp.float32)

    def body(k, _):
        d2 = d2_ref[...]
        m = _tmin0(d2)
        amin = _tmin0(jnp.where(d2 == m, iota0, float(N)))
        selt = (iota0 == amin).astype(jnp.float32)
        d2_ref[...] = jnp.where(iota0 == amin, jnp.inf, d2)
        gct = jax.lax.dot_general(cf, selt, (((0,), (0,)), ((), ())),
                                  preferred_element_type=jnp.float32,
                                  precision=HIGHEST)
        idxf_ref[pl.ds(k, 1), :] = amin + jnp.float32(N) * bf
        gct_ref[k] = gct
        return 0

    jax.lax.fori_loop(0, NSAMPLE, body, 0)

    gxt = gct_ref[:, 0, :]
    gyt = gct_ref[:, 1, :]
    gzt = gct_ref[:, 2, :]
    nit = gxt * gxt + gyt * gyt + gzt * gzt
    gx16 = gxt.astype(jnp.bfloat16).astype(jnp.float32)
    gy16 = gyt.astype(jnp.bfloat16).astype(jnp.float32)
    gz16 = gzt.astype(jnp.bfloat16).astype(jnp.float32)
    crossp = ((gx16[:, None, :] * gx16[None, :, :]
               + gy16[:, None, :] * gy16[None, :, :])
              + gz16[:, None, :] * gz16[None, :, :])
    pd = jnp.clip((nit[:, None, :] + nit[None, :, :]) - 2.0 * crossp,
                  1e-12, None)
    jjj = jax.lax.broadcasted_iota(jnp.int32, (NSAMPLE, NSAMPLE, RB), 0)
    iii = jax.lax.broadcasted_iota(jnp.int32, (NSAMPLE, NSAMPLE, RB), 1)
    pd = jnp.where(jjj == iii, jnp.inf, pd)
    iota_j = jjj.astype(jnp.float32)

    def dbody(_, pdc):
        m = _tmin0(pdc)
        am = _tmin0(jnp.where(pdc == m, iota_j, float(NSAMPLE)))
        return jnp.where(iota_j == am, jnp.inf, pdc)

    pd = jax.lax.fori_loop(0, DENSITY_K - 1, dbody, pd)
    kth = jnp.sqrt(_tmin0(pd).reshape(NSAMPLE, RB))
    raw = jnp.clip(kth, EPS, None)
    raw = raw * raw * raw
    w = raw / jnp.clip(_tsum0(raw), EPS, None)

    ri = jax.lax.broadcasted_iota(jnp.int32, (RB, RB), 0)
    ci = jax.lax.broadcasted_iota(jnp.int32, (RB, RB), 1)
    eye = (ri == ci).astype(jnp.float32)
    idxt = jax.lax.dot_general(eye, idxf_ref[...], (((1,), (1,)), ((), ())),
                               preferred_element_type=jnp.float32,
                               precision=HIGHEST)
    idx_ref[...] = (idxt + 0.5).astype(jnp.int32)
    w_ref[...] = jax.lax.dot_general(eye, w, (((1,), (1,)), ((), ())),
                                     preferred_element_type=jnp.float32,
                                     precision=HIGHEST)


def _tr_kernel(x_ref, o_ref):
    ri = jax.lax.broadcasted_iota(jnp.int32, (C, C), 0)
    ci = jax.lax.broadcasted_iota(jnp.int32, (C, C), 1)
    eye = (ri == ci).astype(jnp.float32)
    o_ref[0] = jax.lax.dot_general(eye, x_ref[0], (((1,), (1,)), ((), ())),
                                   preferred_element_type=jnp.float32,
                                   precision=HIGHEST)


def _make_sc_gather():
    info = plsc.get_sparse_core_info()
    nc = info.num_cores

    mesh = plsc.VectorSubcoreMesh(core_axis_name="c", subcore_axis_name="s")

    @functools.partial(
        pl.kernel, mesh=mesh,
        out_type=jax.ShapeDtypeStruct((4 * N, C), jnp.float32),
        scratch_types=[
            pltpu.VMEM((RB, NSAMPLE), jnp.int32),
            pltpu.VMEM((RB, NSAMPLE), jnp.float32),
            pltpu.VMEM((RB + 16,), jnp.float32),
            pltpu.VMEM((RB + 16,), jnp.float32),
            pltpu.VMEM((RB + 16,), jnp.float32),
            pltpu.VMEM((3, C), jnp.float32),
            pltpu.VMEM((C,), jnp.float32),
            pltpu.VMEM((RB, C), jnp.float32),
            pltpu.VMEM((NSAMPLE, C), jnp.float32),
            pltpu.VMEM((NSAMPLE, C), jnp.float32),
            pltpu.SemaphoreType.DMA,
            pltpu.SemaphoreType.DMA,
        ],
    )
    def sc_gather(t2_hbm, idx_hbm, w_hbm, cx_hbm, cy_hbm, cz_hbm, ws_hbm,
                  sh_hbm, out_hbm, idx_s, w_s, cx_s, cy_s, cz_s, ws_s, sh_s,
                  acc_s, rows_a, rows_b, sem_a, sem_b):
        wid = lax.axis_index("s") * nc + lax.axis_index("c")
        base = wid * RB
        pltpu.sync_copy(idx_hbm.at[wid], idx_s)
        pltpu.sync_copy(w_hbm.at[wid], w_s)
        pltpu.sync_copy(cx_hbm.at[pl.ds(base, RB)], cx_s.at[pl.ds(0, RB)])
        pltpu.sync_copy(cy_hbm.at[pl.ds(base, RB)], cy_s.at[pl.ds(0, RB)])
        pltpu.sync_copy(cz_hbm.at[pl.ds(base, RB)], cz_s.at[pl.ds(0, RB)])
        pltpu.sync_copy(ws_hbm, ws_s)
        pltpu.sync_copy(sh_hbm, sh_s)

        def fire(p, buf, sem):
            pltpu.async_copy(t2_hbm.at[idx_s.at[p, pl.ds(0, NSAMPLE)]],
                             buf, sem)

        def drain(buf, sem):
            pltpu.make_async_copy(t2_hbm.at[pl.ds(0, NSAMPLE)], buf,
                                  sem).wait()

        def compute(p, buf):
            wlo = w_s[p, pl.ds(0, 16)]
            whi = w_s[p, pl.ds(16, 16)]
            cx = cx_s[pl.ds(p, 16)][0] * 10.0
            cy = cy_s[pl.ds(p, 16)][0] * 10.0
            cz = cz_s[pl.ds(p, 16)][0] * 10.0
            negs = []
            accs = []
            for v in range(C // 16):
                sl = pl.ds(v * 16, 16)
                negs.append(sh_s[sl] - (cx * ws_s[0, sl] + cy * ws_s[1, sl]
                                        + cz * ws_s[2, sl]))
                accs.append(jnp.zeros((16,), jnp.float32))
            for k in range(NSAMPLE):
                wk = wlo[k] if k < 16 else whi[k - 16]
                for v in range(C // 16):
                    sl = pl.ds(v * 16, 16)
                    accs[v] = accs[v] + wk * jnp.maximum(
                        buf[k, sl] + negs[v], 0.0)
            for v in range(C // 16):
                acc_s[p, pl.ds(v * 16, 16)] = accs[v]

        fire(0, rows_a, sem_a)
        fire(1, rows_b, sem_b)

        def body(i, _):
            p = 2 * i
            drain(rows_a, sem_a)
            compute(p, rows_a)
            fire(jnp.minimum(p + 2, RB - 1), rows_a, sem_a)
            drain(rows_b, sem_b)
            compute(p + 1, rows_b)
            fire(jnp.minimum(p + 3, RB - 1), rows_b, sem_b)
            return 0

        jax.lax.fori_loop(0, RB // 2, body, 0)
        drain(rows_a, sem_a)
        drain(rows_b, sem_b)
        pltpu.sync_copy(acc_s, out_hbm.at[pl.ds(base, RB)])

    return sc_gather


@jax.jit
def kernel(coords, feats, W, bn_gamma, bn_beta, bn_mean, bn_var):
    B = coords.shape[0]
    scale = bn_gamma / jnp.sqrt(bn_var + BN_EPS)
    shift = bn_beta - bn_mean * scale
    wrp = W[:, :3].T
    wfp = W[:, 3:]
    coords_t = jnp.swapaxes(coords, 1, 2)

    tbl = pl.pallas_call(
        _table_kernel,
        grid=(B,),
        in_specs=[
            pl.BlockSpec((1, C, N), lambda b: (b, 0, 0)),
            pl.BlockSpec((1, N, 3), lambda b: (b, 0, 0)),
            pl.BlockSpec((C, C), lambda b: (0, 0)),
            pl.BlockSpec((3, C), lambda b: (0, 0)),
            pl.BlockSpec((1, C), lambda b: (0, 0)),
        ],
        out_specs=pl.BlockSpec((1, N, C), lambda b: (b, 0, 0)),
        out_shape=jax.ShapeDtypeStruct((B, N, C), jnp.float32),
    )(feats, coords, wfp, wrp, scale[None, :])

    nblk = N // RB
    nw = B * nblk
    idxg, wts = pl.pallas_call(
        _sel_kernel,
        grid=(B, nblk),
        in_specs=[
            pl.BlockSpec((1, 3, RB), lambda b, r: (b, 0, r)),
            pl.BlockSpec((1, N, 3), lambda b, r: (b, 0, 0)),
        ],
        out_specs=[
            pl.BlockSpec((RB, NSAMPLE), lambda b, r: (b * nblk + r, 0)),
            pl.BlockSpec((RB, NSAMPLE), lambda b, r: (b * nblk + r, 0)),
        ],
        out_shape=[
            jax.ShapeDtypeStruct((nw * RB, NSAMPLE), jnp.int32),
            jax.ShapeDtypeStruct((nw * RB, NSAMPLE), jnp.float32),
        ],
        scratch_shapes=[
            pltpu.VMEM((N, RB), jnp.float32),
            pltpu.VMEM((NSAMPLE, 3, RB), jnp.float32),
            pltpu.VMEM((NSAMPLE, RB), jnp.float32),
        ],
    )(coords_t, coords)

    t2 = tbl.reshape(B * N, C)
    cflat = coords.reshape(B * N, 3)
    sc_gather = _make_sc_gather()
    osum = sc_gather(t2, idxg.reshape(nw, RB, NSAMPLE),
                     wts.reshape(nw, RB, NSAMPLE),
                     cflat[:, 0], cflat[:, 1], cflat[:, 2],
                     wrp * scale[None, :], shift)

    out = pl.pallas_call(
        _tr_kernel,
        grid=(B,),
        in_specs=[pl.BlockSpec((1, N, C), lambda b: (b, 0, 0))],
        out_specs=pl.BlockSpec((1, C, N), lambda b: (b, 0, 0)),
        out_shape=jax.ShapeDtypeStruct((B, C, N), jnp.float32),
    )(osum.reshape(B, N, C))
    return out


# speedup vs baseline: 1.0158x; 1.0158x over previous
import functools

import jax
import jax.numpy as jnp
from jax import lax
from jax.experimental import pallas as pl
from jax.experimental.pallas import tpu as pltpu
from jax.experimental.pallas import tpu_sc as plsc

RADIUS = 0.1
NSAMPLE = 32
DENSITY_K = 8
EPS = 1e-08
BN_EPS = 1e-05

N = 2048
C = 128
RB = 256
CH = 64
HIGHEST = jax.lax.Precision.HIGHEST


def _tmin0(x):
    s = x.shape[0]
    while s > 8:
        h = s // 2
        x = jnp.minimum(x[:h], x[h:s])
        s = h
    return jnp.min(x, axis=0, keepdims=True)


def _tsum0(x):
    s = x.shape[0]
    while s > 8:
        h = s // 2
        x = x[:h] + x[h:s]
        s = h
    return jnp.sum(x, axis=0, keepdims=True)


def _table_kernel(feats_ref, coords_ref, wf_ref, wrp_ref, scale_ref, t_ref):
    f = feats_ref[0]
    cf = coords_ref[0]
    wf = wf_ref[...]
    wrp = wrp_ref[...]
    scale = scale_ref[...]
    g = jax.lax.dot_general(f, wf, (((0,), (1,)), ((), ())),
                            preferred_element_type=jnp.float32)
    cfw = jax.lax.dot_general(cf * 10.0, wrp, (((1,), (0,)), ((), ())),
                              preferred_element_type=jnp.float32)
    t_ref[0] = (g + cfw) * scale


def _sel_kernel(ctb_ref, cfull_ref, cf9_ref, idx_ref, w_ref, d2_ref, gct_ref,
                idxf_ref):
    bf = pl.program_id(0).astype(jnp.float32)
    ctb = ctb_ref[0]
    cf = cfull_ref[0]
    cf9 = cf9_ref[0]

    nbt = (ctb[0:1, :] * ctb[0:1, :] + ctb[1:2, :] * ctb[1:2, :]
           + ctb[2:3, :] * ctb[2:3, :])
    cfx = cf[:, 0:1]
    cfy = cf[:, 1:2]
    cfz = cf[:, 2:3]
    nft = cfx * cfx + cfy * cfy + cfz * cfz
    crosst = jax.lax.dot_general(cf, ctb, (((1,), (0,)), ((), ())),
                                 preferred_element_type=jnp.float32)
    d2_ref[...] = jnp.clip((nbt + nft) - 2.0 * crosst, 1e-12, None)

    iota0 = jax.lax.broadcasted_iota(jnp.int32, (N, RB), 0).astype(jnp.float32)

    for k in range(NSAMPLE):
        d2 = d2_ref[...]
        m = _tmin0(d2)
        amin = _tmin0(jnp.where(d2 == m, iota0, float(N)))
        selt = (iota0 == amin).astype(jnp.float32)
        d2_ref[...] = jnp.where(iota0 == amin, jnp.inf, d2)
        gct = jax.lax.dot_general(cf, selt, (((0,), (0,)), ((), ())),
                                  preferred_element_type=jnp.float32,
                                  precision=HIGHEST)
        idxf_ref[k:k + 1, :] = amin + jnp.float32(N) * bf
        gct_ref[k] = gct

    gxt = gct_ref[:, 0, :]
    gyt = gct_ref[:, 1, :]
    gzt = gct_ref[:, 2, :]
    nit = gxt * gxt + gyt * gyt + gzt * gzt
    gx16 = gxt.astype(jnp.bfloat16).astype(jnp.float32)
    gy16 = gyt.astype(jnp.bfloat16).astype(jnp.float32)
    gz16 = gzt.astype(jnp.bfloat16).astype(jnp.float32)
    crossp = ((gx16[:, None, :] * gx16[None, :, :]
               + gy16[:, None, :] * gy16[None, :, :])
              + gz16[:, None, :] * gz16[None, :, :])
    pd = jnp.clip((nit[:, None, :] + nit[None, :, :]) - 2.0 * crossp,
                  1e-12, None)
    jjj = jax.lax.broadcasted_iota(jnp.int32, (NSAMPLE, NSAMPLE, RB), 0)
    iii = jax.lax.broadcasted_iota(jnp.int32, (NSAMPLE, NSAMPLE, RB), 1)
    pd = jnp.where(jjj == iii, jnp.inf, pd)
    iota_j = jjj.astype(jnp.float32)

    def dbody(_, pdc):
        m = _tmin0(pdc)
        am = _tmin0(jnp.where(pdc == m, iota_j, float(NSAMPLE)))
        return jnp.where(iota_j == am, jnp.inf, pdc)

    pd = jax.lax.fori_loop(0, DENSITY_K - 1, dbody, pd)
    kth = jnp.sqrt(_tmin0(pd).reshape(NSAMPLE, RB))
    raw = jnp.clip(kth, EPS, None)
    raw = raw * raw * raw
    w = raw / jnp.clip(_tsum0(raw), EPS, None)

    ri = jax.lax.broadcasted_iota(jnp.int32, (RB, RB), 0)
    ci = jax.lax.broadcasted_iota(jnp.int32, (RB, RB), 1)
    eye = (ri == ci).astype(jnp.float32)
    idxt = jax.lax.dot_general(eye, idxf_ref[...], (((1,), (1,)), ((), ())),
                               preferred_element_type=jnp.float32,
                               precision=HIGHEST)
    idx_ref[...] = (idxt + 0.5).astype(jnp.int32)
    w_ref[...] = jax.lax.dot_general(eye, w, (((1,), (1,)), ((), ())),
                                     preferred_element_type=jnp.float32,
                                     precision=HIGHEST)


def _tr_kernel(x_ref, o_ref):
    ri = jax.lax.broadcasted_iota(jnp.int32, (C, C), 0)
    ci = jax.lax.broadcasted_iota(jnp.int32, (C, C), 1)
    eye = (ri == ci).astype(jnp.float32)
    o_ref[0] = jax.lax.dot_general(eye, x_ref[0], (((1,), (1,)), ((), ())),
                                   preferred_element_type=jnp.float32,
                                   precision=HIGHEST)


def _make_sc_gather():
    info = plsc.get_sparse_core_info()
    nc = info.num_cores

    mesh = plsc.VectorSubcoreMesh(core_axis_name="c", subcore_axis_name="s")

    @functools.partial(
        pl.kernel, mesh=mesh,
        out_type=jax.ShapeDtypeStruct((4 * N, C), jnp.float32),
        scratch_types=[
            pltpu.VMEM((RB, NSAMPLE), jnp.int32),
            pltpu.VMEM((RB, NSAMPLE), jnp.float32),
            pltpu.VMEM((RB + 16,), jnp.float32),
            pltpu.VMEM((RB + 16,), jnp.float32),
            pltpu.VMEM((RB + 16,), jnp.float32),
            pltpu.VMEM((3, C), jnp.float32),
            pltpu.VMEM((C,), jnp.float32),
            pltpu.VMEM((RB, C), jnp.float32),
            pltpu.VMEM((NSAMPLE, C), jnp.float32),
            pltpu.VMEM((NSAMPLE, C), jnp.float32),
            pltpu.SemaphoreType.DMA,
            pltpu.SemaphoreType.DMA,
        ],
    )
    def sc_gather(t2_hbm, idx_hbm, w_hbm, cx_hbm, cy_hbm, cz_hbm, ws_hbm,
                  sh_hbm, out_hbm, idx_s, w_s, cx_s, cy_s, cz_s, ws_s, sh_s,
                  acc_s, rows_a, rows_b, sem_a, sem_b):
        wid = lax.axis_index("s") * nc + lax.axis_index("c")
        base = wid * RB
        pltpu.sync_copy(idx_hbm.at[wid], idx_s)
        pltpu.sync_copy(w_hbm.at[wid], w_s)
        pltpu.sync_copy(cx_hbm.at[pl.ds(base, RB)], cx_s.at[pl.ds(0, RB)])
        pltpu.sync_copy(cy_hbm.at[pl.ds(base, RB)], cy_s.at[pl.ds(0, RB)])
        pltpu.sync_copy(cz_hbm.at[pl.ds(base, RB)], cz_s.at[pl.ds(0, RB)])
        pltpu.sync_copy(ws_hbm, ws_s)
        pltpu.sync_copy(sh_hbm, sh_s)

        def fire(p, buf, sem):
            pltpu.async_copy(t2_hbm.at[idx_s.at[p, pl.ds(0, NSAMPLE)]],
                             buf, sem)

        def drain(buf, sem):
            pltpu.make_async_copy(t2_hbm.at[pl.ds(0, NSAMPLE)], buf,
                                  sem).wait()

        def compute(p, buf):
            wlo = w_s[p, pl.ds(0, 16)]
            whi = w_s[p, pl.ds(16, 16)]
            cx = cx_s[pl.ds(p, 16)][0] * 10.0
            cy = cy_s[pl.ds(p, 16)][0] * 10.0
            cz = cz_s[pl.ds(p, 16)][0] * 10.0
            negs = []
            accs = []
            for v in range(C // 16):
                sl = pl.ds(v * 16, 16)
                negs.append(sh_s[sl] - (cx * ws_s[0, sl] + cy * ws_s[1, sl]
                                        + cz * ws_s[2, sl]))
                accs.append(jnp.zeros((16,), jnp.float32))
            for k in range(NSAMPLE):
                wk = wlo[k] if k < 16 else whi[k - 16]
                for v in range(C // 16):
                    sl = pl.ds(v * 16, 16)
                    accs[v] = accs[v] + wk * jnp.maximum(
                        buf[k, sl] + negs[v], 0.0)
            for v in range(C // 16):
                acc_s[p, pl.ds(v * 16, 16)] = accs[v]

        fire(0, rows_a, sem_a)
        fire(1, rows_b, sem_b)

        def body(i, _):
            p = 2 * i
            drain(rows_a, sem_a)
            compute(p, rows_a)
            fire(jnp.minimum(p + 2, RB - 1), rows_a, sem_a)
            drain(rows_b, sem_b)
            compute(p + 1, rows_b)
            fire(jnp.minimum(p + 3, RB - 1), rows_b, sem_b)
            return 0

        jax.lax.fori_loop(0, RB // 2, body, 0)
        drain(rows_a, sem_a)
        drain(rows_b, sem_b)
        pltpu.sync_copy(acc_s, out_hbm.at[pl.ds(base, RB)])

    return sc_gather


@jax.jit
def kernel(coords, feats, W, bn_gamma, bn_beta, bn_mean, bn_var):
    B = coords.shape[0]
    scale = bn_gamma / jnp.sqrt(bn_var + BN_EPS)
    shift = bn_beta - bn_mean * scale
    wrp = W[:, :3].T
    wfp = W[:, 3:]
    coords_t = jnp.swapaxes(coords, 1, 2)
    chi = coords.astype(jnp.bfloat16).astype(jnp.float32)
    crem = coords - chi
    cmid = crem.astype(jnp.bfloat16).astype(jnp.float32)
    clo = crem - cmid
    cf9 = jnp.concatenate([chi, cmid, clo], axis=-1)

    tbl = pl.pallas_call(
        _table_kernel,
        grid=(B,),
        in_specs=[
            pl.BlockSpec((1, C, N), lambda b: (b, 0, 0)),
            pl.BlockSpec((1, N, 3), lambda b: (b, 0, 0)),
            pl.BlockSpec((C, C), lambda b: (0, 0)),
            pl.BlockSpec((3, C), lambda b: (0, 0)),
            pl.BlockSpec((1, C), lambda b: (0, 0)),
        ],
        out_specs=pl.BlockSpec((1, N, C), lambda b: (b, 0, 0)),
        out_shape=jax.ShapeDtypeStruct((B, N, C), jnp.float32),
    )(feats, coords, wfp, wrp, scale[None, :])

    nblk = N // RB
    nw = B * nblk
    idxg, wts = pl.pallas_call(
        _sel_kernel,
        grid=(B, nblk),
        in_specs=[
            pl.BlockSpec((1, 3, RB), lambda b, r: (b, 0, r)),
            pl.BlockSpec((1, N, 3), lambda b, r: (b, 0, 0)),
            pl.BlockSpec((1, N, 9), lambda b, r: (b, 0, 0)),
        ],
        out_specs=[
            pl.BlockSpec((RB, NSAMPLE), lambda b, r: (b * nblk + r, 0)),
            pl.BlockSpec((RB, NSAMPLE), lambda b, r: (b * nblk + r, 0)),
        ],
        out_shape=[
            jax.ShapeDtypeStruct((nw * RB, NSAMPLE), jnp.int32),
            jax.ShapeDtypeStruct((nw * RB, NSAMPLE), jnp.float32),
        ],
        scratch_shapes=[
            pltpu.VMEM((N, RB), jnp.float32),
            pltpu.VMEM((NSAMPLE, 3, RB), jnp.float32),
            pltpu.VMEM((NSAMPLE, RB), jnp.float32),
        ],
    )(coords_t, coords, cf9)

    t2 = tbl.reshape(B * N, C)
    cflat = coords.reshape(B * N, 3)
    sc_gather = _make_sc_gather()
    osum = sc_gather(t2, idxg.reshape(nw, RB, NSAMPLE),
                     wts.reshape(nw, RB, NSAMPLE),
                     cflat[:, 0], cflat[:, 1], cflat[:, 2],
                     wrp * scale[None, :], shift)

    out = pl.pallas_call(
        _tr_kernel,
        grid=(B,),
        in_specs=[pl.BlockSpec((1, N, C), lambda b: (b, 0, 0))],
        out_specs=pl.BlockSpec((1, C, N), lambda b: (b, 0, 0)),
        out_shape=jax.ShapeDtypeStruct((B, C, N), jnp.float32),
    )(osum.reshape(B, N, C))
    return out


# speedup vs baseline: 1.0168x; 1.0011x over previous
import functools

import jax
import jax.numpy as jnp
from jax import lax
from jax.experimental import pallas as pl
from jax.experimental.pallas import tpu as pltpu
from jax.experimental.pallas import tpu_sc as plsc

RADIUS = 0.1
NSAMPLE = 32
DENSITY_K = 8
EPS = 1e-08
BN_EPS = 1e-05

N = 2048
C = 128
RB = 256
CH = 64
HIGHEST = jax.lax.Precision.HIGHEST


def _tmin0(x):
    s = x.shape[0]
    while s > 8:
        h = s // 2
        x = jnp.minimum(x[:h], x[h:s])
        s = h
    return jnp.min(x, axis=0, keepdims=True)


def _tsum0(x):
    s = x.shape[0]
    while s > 8:
        h = s // 2
        x = x[:h] + x[h:s]
        s = h
    return jnp.sum(x, axis=0, keepdims=True)


def _table_kernel(feats_ref, coords_ref, wf_ref, wrp_ref, scale_ref, t_ref):
    f = feats_ref[0]
    cf = coords_ref[0]
    wf = wf_ref[...]
    wrp = wrp_ref[...]
    scale = scale_ref[...]
    g = jax.lax.dot_general(f, wf, (((0,), (1,)), ((), ())),
                            preferred_element_type=jnp.float32)
    cfw = jax.lax.dot_general(cf * 10.0, wrp, (((1,), (0,)), ((), ())),
                              preferred_element_type=jnp.float32)
    t_ref[0] = (g + cfw) * scale


def _sel_kernel(ctb_ref, cfull_ref, idx_ref, w_ref, d2_ref, gct_ref,
                idxf_ref):
    bf = pl.program_id(0).astype(jnp.float32)
    ctb = ctb_ref[0]
    cf = cfull_ref[0]

    nbt = (ctb[0:1, :] * ctb[0:1, :] + ctb[1:2, :] * ctb[1:2, :]
           + ctb[2:3, :] * ctb[2:3, :])
    cfx = cf[:, 0:1]
    cfy = cf[:, 1:2]
    cfz = cf[:, 2:3]
    nft = cfx * cfx + cfy * cfy + cfz * cfz
    crosst = jax.lax.dot_general(cf, ctb, (((1,), (0,)), ((), ())),
                                 preferred_element_type=jnp.float32)
    d2_ref[...] = jnp.clip((nbt + nft) - 2.0 * crosst, 1e-12, None)

    iota0 = jax.lax.broadcasted_iota(jnp.int32, (N, RB), 0).astype(jnp.float32)

    for k in range(NSAMPLE):
        d2 = d2_ref[...]
        m = _tmin0(d2)
        amin = _tmin0(jnp.where(d2 == m, iota0, float(N)))
        selt = (iota0 == amin).astype(jnp.float32)
        d2_ref[...] = jnp.where(iota0 == amin, jnp.inf, d2)
        gct = jax.lax.dot_general(cf, selt, (((0,), (0,)), ((), ())),
                                  preferred_element_type=jnp.float32,
                                  precision=HIGHEST)
        idxf_ref[k:k + 1, :] = amin + jnp.float32(N) * bf
        gct_ref[k] = gct

    gxt = gct_ref[:, 0, :]
    gyt = gct_ref[:, 1, :]
    gzt = gct_ref[:, 2, :]
    nit = gxt * gxt + gyt * gyt + gzt * gzt
    gx16 = gxt.astype(jnp.bfloat16).astype(jnp.float32)
    gy16 = gyt.astype(jnp.bfloat16).astype(jnp.float32)
    gz16 = gzt.astype(jnp.bfloat16).astype(jnp.float32)
    crossp = ((gx16[:, None, :] * gx16[None, :, :]
               + gy16[:, None, :] * gy16[None, :, :])
              + gz16[:, None, :] * gz16[None, :, :])
    pd = jnp.clip((nit[:, None, :] + nit[None, :, :]) - 2.0 * crossp,
                  1e-12, None)
    jjj = jax.lax.broadcasted_iota(jnp.int32, (NSAMPLE, NSAMPLE, RB), 0)
    iii = jax.lax.broadcasted_iota(jnp.int32, (NSAMPLE, NSAMPLE, RB), 1)
    pd = jnp.where(jjj == iii, jnp.inf, pd)
    iota_j = jjj.astype(jnp.float32)

    def dbody(_, pdc):
        m = _tmin0(pdc)
        am = _tmin0(jnp.where(pdc == m, iota_j, float(NSAMPLE)))
        return jnp.where(iota_j == am, jnp.inf, pdc)

    pd = jax.lax.fori_loop(0, DENSITY_K - 1, dbody, pd)
    kth = jnp.sqrt(_tmin0(pd).reshape(NSAMPLE, RB))
    raw = jnp.clip(kth, EPS, None)
    raw = raw * raw * raw
    w = raw / jnp.clip(_tsum0(raw), EPS, None)

    ri = jax.lax.broadcasted_iota(jnp.int32, (RB, RB), 0)
    ci = jax.lax.broadcasted_iota(jnp.int32, (RB, RB), 1)
    eye = (ri == ci).astype(jnp.float32)
    idxt = jax.lax.dot_general(eye, idxf_ref[...], (((1,), (1,)), ((), ())),
                               preferred_element_type=jnp.float32,
                               precision=HIGHEST)
    idx_ref[...] = (idxt + 0.5).astype(jnp.int32)
    w_ref[...] = jax.lax.dot_general(eye, w, (((1,), (1,)), ((), ())),
                                     preferred_element_type=jnp.float32,
                                     precision=HIGHEST)


def _tr_kernel(x_ref, o_ref):
    ri = jax.lax.broadcasted_iota(jnp.int32, (C, C), 0)
    ci = jax.lax.broadcasted_iota(jnp.int32, (C, C), 1)
    eye = (ri == ci).astype(jnp.float32)
    o_ref[0] = jax.lax.dot_general(eye, x_ref[0], (((1,), (1,)), ((), ())),
                                   preferred_element_type=jnp.float32,
                                   precision=HIGHEST)


def _make_sc_gather():
    info = plsc.get_sparse_core_info()
    nc = info.num_cores

    mesh = plsc.VectorSubcoreMesh(core_axis_name="c", subcore_axis_name="s")

    @functools.partial(
        pl.kernel, mesh=mesh,
        out_type=jax.ShapeDtypeStruct((4 * N, C), jnp.float32),
        scratch_types=[
            pltpu.VMEM((RB, NSAMPLE), jnp.int32),
            pltpu.VMEM((RB, NSAMPLE), jnp.float32),
            pltpu.VMEM((RB + 16,), jnp.float32),
            pltpu.VMEM((RB + 16,), jnp.float32),
            pltpu.VMEM((RB + 16,), jnp.float32),
            pltpu.VMEM((3, C), jnp.float32),
            pltpu.VMEM((C,), jnp.float32),
            pltpu.VMEM((RB, C), jnp.float32),
            pltpu.VMEM((NSAMPLE, C), jnp.float32),
            pltpu.VMEM((NSAMPLE, C), jnp.float32),
            pltpu.SemaphoreType.DMA,
            pltpu.SemaphoreType.DMA,
        ],
    )
    def sc_gather(t2_hbm, idx_hbm, w_hbm, cx_hbm, cy_hbm, cz_hbm, ws_hbm,
                  sh_hbm, out_hbm, idx_s, w_s, cx_s, cy_s, cz_s, ws_s, sh_s,
                  acc_s, rows_a, rows_b, sem_a, sem_b):
        wid = lax.axis_index("s") * nc + lax.axis_index("c")
        base = wid * RB
        pltpu.sync_copy(idx_hbm.at[wid], idx_s)
        pltpu.sync_copy(w_hbm.at[wid], w_s)
        pltpu.sync_copy(cx_hbm.at[pl.ds(base, RB)], cx_s.at[pl.ds(0, RB)])
        pltpu.sync_copy(cy_hbm.at[pl.ds(base, RB)], cy_s.at[pl.ds(0, RB)])
        pltpu.sync_copy(cz_hbm.at[pl.ds(base, RB)], cz_s.at[pl.ds(0, RB)])
        pltpu.sync_copy(ws_hbm, ws_s)
        pltpu.sync_copy(sh_hbm, sh_s)

        def fire(p, buf, sem):
            pltpu.async_copy(t2_hbm.at[idx_s.at[p, pl.ds(0, NSAMPLE)]],
                             buf, sem)

        def drain(buf, sem):
            pltpu.make_async_copy(t2_hbm.at[pl.ds(0, NSAMPLE)], buf,
                                  sem).wait()

        def compute(p, buf):
            wlo = w_s[p, pl.ds(0, 16)]
            whi = w_s[p, pl.ds(16, 16)]
            cx = cx_s[pl.ds(p, 16)][0] * 10.0
            cy = cy_s[pl.ds(p, 16)][0] * 10.0
            cz = cz_s[pl.ds(p, 16)][0] * 10.0
            negs = []
            accs = []
            for v in range(C // 16):
                sl = pl.ds(v * 16, 16)
                negs.append(sh_s[sl] - (cx * ws_s[0, sl] + cy * ws_s[1, sl]
                                        + cz * ws_s[2, sl]))
                accs.append(jnp.zeros((16,), jnp.float32))
            for k in range(NSAMPLE):
                wk = wlo[k] if k < 16 else whi[k - 16]
                for v in range(C // 16):
                    sl = pl.ds(v * 16, 16)
                    accs[v] = accs[v] + wk * jnp.maximum(
                        buf[k, sl] + negs[v], 0.0)
            for v in range(C // 16):
                acc_s[p, pl.ds(v * 16, 16)] = accs[v]

        fire(0, rows_a, sem_a)
        fire(1, rows_b, sem_b)

        def body(i, _):
            p = 2 * i
            drain(rows_a, sem_a)
            compute(p, rows_a)
            fire(jnp.minimum(p + 2, RB - 1), rows_a, sem_a)
            drain(rows_b, sem_b)
            compute(p + 1, rows_b)
            fire(jnp.minimum(p + 3, RB - 1), rows_b, sem_b)
            return 0

        jax.lax.fori_loop(0, RB // 2, body, 0)
        drain(rows_a, sem_a)
        drain(rows_b, sem_b)
        pltpu.sync_copy(acc_s, out_hbm.at[pl.ds(base, RB)])

    return sc_gather


@jax.jit
def kernel(coords, feats, W, bn_gamma, bn_beta, bn_mean, bn_var):
    B = coords.shape[0]
    scale = bn_gamma / jnp.sqrt(bn_var + BN_EPS)
    shift = bn_beta - bn_mean * scale
    wrp = W[:, :3].T
    wfp = W[:, 3:]
    coords_t = jnp.swapaxes(coords, 1, 2)

    tbl = pl.pallas_call(
        _table_kernel,
        grid=(B,),
        in_specs=[
            pl.BlockSpec((1, C, N), lambda b: (b, 0, 0)),
            pl.BlockSpec((1, N, 3), lambda b: (b, 0, 0)),
            pl.BlockSpec((C, C), lambda b: (0, 0)),
            pl.BlockSpec((3, C), lambda b: (0, 0)),
            pl.BlockSpec((1, C), lambda b: (0, 0)),
        ],
        out_specs=pl.BlockSpec((1, N, C), lambda b: (b, 0, 0)),
        out_shape=jax.ShapeDtypeStruct((B, N, C), jnp.float32),
    )(feats, coords, wfp, wrp, scale[None, :])

    nblk = N // RB
    nw = B * nblk
    idxg, wts = pl.pallas_call(
        _sel_kernel,
        grid=(B, nblk),
        in_specs=[
            pl.BlockSpec((1, 3, RB), lambda b, r: (b, 0, r)),
            pl.BlockSpec((1, N, 3), lambda b, r: (b, 0, 0)),
        ],
        out_specs=[
            pl.BlockSpec((RB, NSAMPLE), lambda b, r: (b * nblk + r, 0)),
            pl.BlockSpec((RB, NSAMPLE), lambda b, r: (b * nblk + r, 0)),
        ],
        out_shape=[
            jax.ShapeDtypeStruct((nw * RB, NSAMPLE), jnp.int32),
            jax.ShapeDtypeStruct((nw * RB, NSAMPLE), jnp.float32),
        ],
        scratch_shapes=[
            pltpu.VMEM((N, RB), jnp.float32),
            pltpu.VMEM((NSAMPLE, 3, RB), jnp.float32),
            pltpu.VMEM((NSAMPLE, RB), jnp.float32),
        ],
    )(coords_t, coords)

    t2 = tbl.reshape(B * N, C)
    cflat = coords.reshape(B * N, 3)
    sc_gather = _make_sc_gather()
    osum = sc_gather(t2, idxg.reshape(nw, RB, NSAMPLE),
                     wts.reshape(nw, RB, NSAMPLE),
                     cflat[:, 0], cflat[:, 1], cflat[:, 2],
                     wrp * scale[None, :], shift)

    out = pl.pallas_call(
        _tr_kernel,
        grid=(B,),
        in_specs=[pl.BlockSpec((1, N, C), lambda b: (b, 0, 0))],
        out_specs=pl.BlockSpec((1, C, N), lambda b: (b, 0, 0)),
        out_shape=jax.ShapeDtypeStruct((B, C, N), jnp.float32),
    )(osum.reshape(B, N, C))
    return out
